# baseline (speedup 1.0000x reference)
import functools

import jax
import jax.numpy as jnp
from jax import lax
from jax.experimental import pallas as pl
from jax.experimental.pallas import tpu as pltpu
from jax.experimental.pallas import tpu_sc as plsc

_F32 = jnp.float32
_I32 = jnp.int32

N = 10000
E = 320000
D = 128
NW = 32
CH = 128
NCH = 79
EPW = NCH * CH
E_PAD = NW * EPW
NROW = 640
NPAD = NROW * 16


def _proj_body(x_ref, w_ref, as_ref, ad_ref, h_ref, asr_ref, ads_ref):
    h = jnp.dot(x_ref[...], w_ref[...], preferred_element_type=_F32)
    h_ref[...] = h
    asr_ref[...] = jnp.sum(h * as_ref[...], axis=1, keepdims=True)
    ads_ref[...] = jnp.sum(h * ad_ref[...], axis=1, keepdims=True)


def _tc_proj(x, W, a_s, a_d):
    BN = 1000
    return pl.pallas_call(
        _proj_body,
        grid=(N // BN,),
        in_specs=[pl.BlockSpec((BN, D), lambda i: (i, 0)),
                  pl.BlockSpec((D, D), lambda i: (0, 0)),
                  pl.BlockSpec((1, D), lambda i: (0, 0)),
                  pl.BlockSpec((1, D), lambda i: (0, 0))],
        out_specs=[pl.BlockSpec((BN, D), lambda i: (i, 0)),
                   pl.BlockSpec((BN, 1), lambda i: (i, 0)),
                   pl.BlockSpec((BN, 1), lambda i: (i, 0))],
        out_shape=[jax.ShapeDtypeStruct((N, D), _F32),
                   jax.ShapeDtypeStruct((N, 1), _F32),
                   jax.ShapeDtypeStruct((N, 1), _F32)],
    )(x, W, a_s.reshape(1, D), a_d.reshape(1, D))


def _den1_body(c_ref, dp_ref, deg_ref, sew_ref, asr_ref, ads_ref,
               la_ref, rden_ref):
    c = c_ref[0, 0]
    p = dp_ref[0:80, :] + dp_ref[80:160, :]
    deg = deg_ref[0:80, :] + deg_ref[80:160, :]
    sew = sew_ref[0:80, :] + sew_ref[80:160, :]
    la = sew / jnp.maximum(deg, 1.0)
    tl = asr_ref[...] + ads_ref[...] + c * la
    wl = jnp.exp(jnp.where(tl >= 0, tl, 0.2 * tl))
    la_ref[...] = la
    rden_ref[...] = 1.0 / (p + wl)


def _tc_den1(c, denp, deg, sew, asr_pad, ads_pad):
    full = lambda r: pl.BlockSpec((r, D), lambda: (0, 0))
    return pl.pallas_call(
        _den1_body,
        in_specs=[pl.BlockSpec((1, 1), lambda: (0, 0)),
                  full(160), full(160), full(160), full(80), full(80)],
        out_specs=[full(80), full(80)],
        out_shape=[jax.ShapeDtypeStruct((80, D), _F32),
                   jax.ShapeDtypeStruct((80, D), _F32)],
    )(c.reshape(1, 1), denp, deg, sew, asr_pad, ads_pad)


def _den2_body(c_ref, dp_ref, la_ref, asr_ref, ads_ref, rden_ref):
    c = c_ref[0, 0]
    p = dp_ref[0:80, :] + dp_ref[80:160, :]
    tl = asr_ref[...] + ads_ref[...] + c * la_ref[...]
    wl = jnp.exp(jnp.where(tl >= 0, tl, 0.2 * tl))
    rden_ref[...] = 1.0 / (p + wl)


def _tc_den2(c, denp, la, asr_pad, ads_pad):
    full = lambda r: pl.BlockSpec((r, D), lambda: (0, 0))
    return pl.pallas_call(
        _den2_body,
        in_specs=[pl.BlockSpec((1, 1), lambda: (0, 0)),
                  full(160), full(80), full(80), full(80)],
        out_specs=full(80),
        out_shape=jax.ShapeDtypeStruct((80, D), _F32),
    )(c.reshape(1, 1), denp, la, asr_pad, ads_pad)


def _mid_body(c_ref, p0_ref, p1_ref, h_ref, asr_ref, ads_ref, la_ref,
              rden_ref, w2_ref, as2_ref, ad2_ref, b1_ref,
              g_ref, asr2_ref, ads2_ref):
    c = c_ref[0, 0]
    tl = asr_ref[...] + ads_ref[...] + c * la_ref[...]
    wl = jnp.exp(jnp.where(tl >= 0, tl, 0.2 * tl))
    al = wl * rden_ref[...]
    out1 = p0_ref[...] + p1_ref[...] + al * h_ref[...]
    h2 = out1 + b1_ref[...]
    h2 = jnp.where(h2 > 0, h2, jnp.exp(jnp.minimum(h2, 0.0)) - 1.0)
    g = jnp.dot(h2, w2_ref[...], preferred_element_type=_F32)
    g_ref[...] = g
    asr2_ref[...] = jnp.sum(g * as2_ref[...], axis=1, keepdims=True)
    ads2_ref[...] = jnp.sum(g * ad2_ref[...], axis=1, keepdims=True)


def _tc_mid(c1, p0, p1, h, asr, ads, la, rden, W2, a_s2, a_d2, b1):
    BN = 1000
    vec = lambda: pl.BlockSpec((BN, 1), lambda i: (i, 0))
    mat = lambda: pl.BlockSpec((BN, D), lambda i: (i, 0))
    full = lambda r: pl.BlockSpec((r, D), lambda i: (0, 0))
    return pl.pallas_call(
        _mid_body,
        grid=(N // BN,),
        in_specs=[pl.BlockSpec((1, 1), lambda i: (0, 0)),
                  mat(), mat(), mat(), vec(), vec(), vec(), vec(),
                  full(D), full(1), full(1), full(1)],
        out_specs=[mat(), vec(), vec()],
        out_shape=[jax.ShapeDtypeStruct((N, D), _F32),
                   jax.ShapeDtypeStruct((N, 1), _F32),
                   jax.ShapeDtypeStruct((N, 1), _F32)],
    )(c1.reshape(1, 1), p0, p1, h, asr, ads, la, rden,
      W2, a_s2.reshape(1, D), a_d2.reshape(1, D), b1.reshape(1, D))


def _fin_body(c_ref, p0_ref, p1_ref, g_ref, asr_ref, ads_ref, la_ref,
              rden_ref, b2_ref, o_ref):
    c = c_ref[0, 0]
    tl = asr_ref[...] + ads_ref[...] + c * la_ref[...]
    wl = jnp.exp(jnp.where(tl >= 0, tl, 0.2 * tl))
    al = wl * rden_ref[...]
    o_ref[...] = p0_ref[...] + p1_ref[...] + al * g_ref[...] + b2_ref[...]


def _tc_fin(c2, p0, p1, g, asr2, ads2, la, rden2, b2):
    BN = 1000
    vec = lambda: pl.BlockSpec((BN, 1), lambda i: (i, 0))
    mat = lambda: pl.BlockSpec((BN, D), lambda i: (i, 0))
    return pl.pallas_call(
        _fin_body,
        grid=(N // BN,),
        in_specs=[pl.BlockSpec((1, 1), lambda i: (0, 0)),
                  mat(), mat(), mat(), vec(), vec(), vec(), vec(),
                  pl.BlockSpec((1, D), lambda i: (0, 0))],
        out_specs=mat(),
        out_shape=jax.ShapeDtypeStruct((N, D), _F32),
    )(c2.reshape(1, 1), p0, p1, g, asr2, ads2, la, rden2, b2.reshape(1, D))


_SC_MESH = plsc.VectorSubcoreMesh(core_axis_name="c", subcore_axis_name="s")
_SC_PARAMS = pltpu.CompilerParams(needs_layout_passes=False,
                                  use_tc_tiling_on_sc=False)

_GATHER_DNUMS = lax.GatherDimensionNumbers(
    offset_dims=(), collapsed_slice_dims=(0,), start_index_map=(0,))


def _bcast_lane(v, j):
    idx = jnp.full((16, 1), j, _I32)
    return lax.gather(v, idx, _GATHER_DNUMS, slice_sizes=(1,),
                      mode=lax.GatherScatterMode.PROMISE_IN_BOUNDS)


def _p1_body(src_hbm, dst_hbm, ew_hbm, asr_hbm, ads_hbm, cv_hbm,
             w_hbm, denp_hbm, degp_hbm, sewp_hbm,
             asr_v, ads_v, den_v, deg_v, sew_v,
             src_w, dst_w, ew_w, w_w, cv_v, idx_v,
             den_sh, deg_sh, sew_sh):
    cid = lax.axis_index("c")
    sid = lax.axis_index("s")
    wid = sid * 2 + cid
    pltpu.sync_copy(asr_hbm, asr_v)
    pltpu.sync_copy(ads_hbm, ads_v)
    pltpu.sync_copy(cv_hbm, cv_v)
    pltpu.sync_copy(src_hbm.at[wid], src_w)
    pltpu.sync_copy(dst_hbm.at[wid], dst_w)
    pltpu.sync_copy(ew_hbm.at[wid], ew_w)
    lanes = lax.iota(_I32, 16)
    for j in range(5):
        for k in range(8):
            idx_v[j, pl.ds(k * 16, 16)] = j * 128 + k * 16 + lanes

    def zbody(r, carry):
        z = jnp.zeros((16,), _F32)
        den_v[r, :] = z
        deg_v[r, :] = z
        sew_v[r, :] = z
        return carry
    lax.fori_loop(0, NROW, zbody, 0)

    @pl.when(sid == 0)
    def _():
        pltpu.sync_copy(den_v, den_sh)
        pltpu.sync_copy(deg_v, deg_sh)
        pltpu.sync_copy(sew_v, sew_sh)
    plsc.subcore_barrier()

    cv16 = cv_v[...]
    ones = jnp.ones((16,), _F32)
    ebase = wid * EPW

    def chunk(ch, carry):
        for g in range(8):
            sl = pl.ds(g * 16, 16)
            srcs = src_w[ch, sl]
            dsts = dst_w[ch, sl]
            ewv = ew_w[ch, sl]
            rows = lax.shift_right_logical(dsts, 4)
            cols = jnp.bitwise_and(dsts, 15)
            asrv = plsc.load_gather(
                asr_v, [lax.shift_right_logical(srcs, 4),
                        jnp.bitwise_and(srcs, 15)])
            adsv = plsc.load_gather(ads_v, [rows, cols])
            t = asrv + adsv + cv16 * ewv
            t = jnp.where(t >= 0, t, 0.2 * t)
            w = jnp.exp(t)
            gid = ebase + ch * CH + g * 16 + lanes
            m = gid < E
            w = jnp.where(m, w, 0.0)
            w_w[ch, sl] = w
            plsc.addupdate_scatter(den_v, [rows, cols], w, mask=m)
            plsc.addupdate_scatter(deg_v, [rows, cols], ones, mask=m)
            plsc.addupdate_scatter(sew_v, [rows, cols], ewv, mask=m)
        return carry
    lax.fori_loop(0, NCH, chunk, 0)
    pltpu.sync_copy(w_w, w_hbm.at[wid])
    plsc.subcore_barrier()
    for j in range(5):
        s = pl.ds(j * 128, 128)
        pltpu.sync_copy(den_v.at[s], den_sh.at[idx_v.at[j]], add=True)
        pltpu.sync_copy(deg_v.at[s], deg_sh.at[idx_v.at[j]], add=True)
        pltpu.sync_copy(sew_v.at[s], sew_sh.at[idx_v.at[j]], add=True)
    plsc.subcore_barrier()

    @pl.when(sid == 0)
    def _():
        pltpu.sync_copy(den_sh, denp_hbm.at[cid])
        pltpu.sync_copy(deg_sh, degp_hbm.at[cid])
        pltpu.sync_copy(sew_sh, sewp_hbm.at[cid])


def _sc_pass1(src3, dst3, ew3, asr, ads, cv):
    out = pl.kernel(
        _p1_body,
        out_type=[jax.ShapeDtypeStruct((NW, NCH, CH), _F32),
                  jax.ShapeDtypeStruct((2, NROW, 16), _F32),
                  jax.ShapeDtypeStruct((2, NROW, 16), _F32),
                  jax.ShapeDtypeStruct((2, NROW, 16), _F32)],
        mesh=_SC_MESH,
        compiler_params=_SC_PARAMS,
        scratch_types=[
            pltpu.VMEM((NROW, 16), _F32), pltpu.VMEM((NROW, 16), _F32),
            pltpu.VMEM((NROW, 16), _F32), pltpu.VMEM((NROW, 16), _F32),
            pltpu.VMEM((NROW, 16), _F32),
            pltpu.VMEM((NCH, CH), _I32), pltpu.VMEM((NCH, CH), _I32),
            pltpu.VMEM((NCH, CH), _F32), pltpu.VMEM((NCH, CH), _F32),
            pltpu.VMEM((16,), _F32), pltpu.VMEM((5, 128), _I32),
            pltpu.VMEM_SHARED((NROW, 16), _F32),
            pltpu.VMEM_SHARED((NROW, 16), _F32),
            pltpu.VMEM_SHARED((NROW, 16), _F32),
        ],
    )(src3, dst3, ew3, asr, ads, cv)
    return out


def _p2_body(e_hbm, rden_hbm, h_hbm,
             outp_hbm,
             rden_v, edata, rows2, sem_g,
             out_sh):
    cid = lax.axis_index("c")
    sid = lax.axis_index("s")
    wid = sid * 2 + cid
    pltpu.sync_copy(rden_hbm, rden_v)

    def zb(r, carry):
        for k in range(8):
            rows2[0, r, pl.ds(k * 16, 16)] = jnp.zeros((16,), _F32)
        return carry
    lax.fori_loop(0, CH, zb, 0)
    for t in range(5):
        pltpu.sync_copy(rows2.at[0],
                        out_sh.at[pl.ds(sid * 640 + t * 128, 128)])
    plsc.subcore_barrier()

    pltpu.sync_copy(e_hbm.at[wid, 0], edata.at[0])
    pltpu.async_copy(h_hbm.at[edata.at[0, 0]], rows2.at[0], sem_g.at[0])

    def cb(ch, carry):
        b = jnp.bitwise_and(ch, 1)
        nb = 1 - b

        @pl.when(ch + 1 < NCH)
        def _():
            pltpu.sync_copy(e_hbm.at[wid, ch + 1], edata.at[nb])
            pltpu.async_copy(h_hbm.at[edata.at[nb, 0]], rows2.at[nb],
                             sem_g.at[nb])
        pltpu.make_async_copy(h_hbm.at[edata.at[b, 0]], rows2.at[b],
                              sem_g.at[b]).wait()

        def gb(g, carry2):
            sl = pl.ds(g * 16, 16)
            dsts = edata[b, 1, sl]
            rws = lax.shift_right_logical(dsts, 4)
            cls = jnp.bitwise_and(dsts, 15)
            wv = plsc.bitcast(edata[b, 2, sl], _F32)
            av = wv * plsc.load_gather(rden_v, [rws, cls])
            for j in range(16):
                bc = _bcast_lane(av, j)
                e = g * 16 + j
                for k in range(8):
                    sk = pl.ds(k * 16, 16)
                    rows2[b, e, sk] = rows2[b, e, sk] * bc
            return carry2
        lax.fori_loop(0, 8, gb, 0)
        pltpu.sync_copy(rows2.at[b], out_sh.at[edata.at[b, 1]], add=True)
        return carry
    lax.fori_loop(0, NCH, cb, 0)
    plsc.subcore_barrier()
    s = pl.ds(sid * 640, 640)
    pltpu.sync_copy(out_sh.at[s], outp_hbm.at[cid, s])


def _sc_pass2(edata, rden, h):
    return pl.kernel(
        _p2_body,
        out_type=jax.ShapeDtypeStruct((2, NPAD, D), _F32),
        mesh=_SC_MESH,
        compiler_params=_SC_PARAMS,
        scratch_types=[
            pltpu.VMEM((NROW, 16), _F32),
            pltpu.VMEM((2, 3, CH), _I32),
            pltpu.VMEM((2, CH, D), _F32),
            pltpu.SemaphoreType.DMA((2,)),
            pltpu.VMEM_SHARED((NPAD, D), _F32),
        ],
    )(edata, rden, h)


def kernel(x, edge_index, edge_weight, W1, as1, ad1, We1, ae1, b1,
           W2, as2, ad2, We2, ae2, b2):
    src, dst = edge_index[0], edge_index[1]
    ew = edge_weight[:, 0]
    c1 = jnp.sum(We1.reshape(-1) * ae1.reshape(-1))
    c2 = jnp.sum(We2.reshape(-1) * ae2.reshape(-1))
    cv1 = jnp.full((16,), c1, _F32)
    cv2 = jnp.full((16,), c2, _F32)

    pad_i = jnp.zeros((E_PAD - E,), _I32)
    pad_f = jnp.zeros((E_PAD - E,), _F32)
    src3 = jnp.concatenate([src, pad_i]).reshape(NW, NCH, CH)
    dst3 = jnp.concatenate([dst, pad_i]).reshape(NW, NCH, CH)
    ew3 = jnp.concatenate([ew, pad_f]).reshape(NW, NCH, CH)
    padn = jnp.zeros((NPAD - N,), _F32)

    h, asr1, ads1 = _tc_proj(x, W1, as1, ad1)
    asr1pad = jnp.concatenate([asr1.reshape(N), padn])
    ads1pad = jnp.concatenate([ads1.reshape(N), padn])
    w1, denp1, degp, sewp = _sc_pass1(src3, dst3, ew3,
                                      asr1pad.reshape(NROW, 16),
                                      ads1pad.reshape(NROW, 16), cv1)
    asr1p = asr1pad.reshape(80, D)
    ads1p = ads1pad.reshape(80, D)
    la80, rden1_80 = _tc_den1(c1, denp1.reshape(160, D),
                              degp.reshape(160, D), sewp.reshape(160, D),
                              asr1p, ads1p)
    rden1_640 = rden1_80.reshape(NROW, 16)
    edata1 = jnp.stack(
        [src3, dst3, lax.bitcast_convert_type(w1, _I32)], axis=2)
    outp1 = _sc_pass2(edata1, rden1_640, h)
    la = la80.reshape(NPAD)[:N].reshape(N, 1)
    rden1 = rden1_80.reshape(NPAD)[:N].reshape(N, 1)

    g, asr2, ads2 = _tc_mid(c1, outp1[0, :N], outp1[1, :N], h,
                            asr1, ads1, la, rden1, W2, as2, ad2, b1)

    asr2pad = jnp.concatenate([asr2.reshape(N), padn])
    ads2pad = jnp.concatenate([ads2.reshape(N), padn])
    w2e, denp2, _, _ = _sc_pass1(src3, dst3, ew3,
                                 asr2pad.reshape(NROW, 16),
                                 ads2pad.reshape(NROW, 16), cv2)
    asr2p = asr2pad.reshape(80, D)
    ads2p = ads2pad.reshape(80, D)
    rden2_80 = _tc_den2(c2, denp2.reshape(160, D), la80, asr2p, ads2p)
    edata2 = jnp.stack(
        [src3, dst3, lax.bitcast_convert_type(w2e, _I32)], axis=2)
    outp2 = _sc_pass2(edata2, rden2_80.reshape(NROW, 16), g)
    rden2 = rden2_80.reshape(NPAD)[:N].reshape(N, 1)

    o = _tc_fin(c2, outp2[0, :N], outp2[1, :N], g, asr2, ads2,
                la, rden2, b2)
    return o


# speedup vs baseline: 1.1701x; 1.1701x over previous
import functools

import jax
import jax.numpy as jnp
from jax import lax
from jax.experimental import pallas as pl
from jax.experimental.pallas import tpu as pltpu
from jax.experimental.pallas import tpu_sc as plsc

_F32 = jnp.float32
_I32 = jnp.int32

N = 10000
E = 320000
D = 128
NW = 32
CH = 128
NCH = 80
EPW = NCH * CH
E_PAD = NW * EPW
NROW = 640
NPAD = NROW * 16


def _proj_body(x_ref, w_ref, as_ref, ad_ref, h_ref, asr_ref, ads_ref):
    h = jnp.dot(x_ref[...], w_ref[...], preferred_element_type=_F32)
    h_ref[...] = h
    asr_ref[...] = jnp.sum(h * as_ref[...], axis=1, keepdims=True)
    ads_ref[...] = jnp.sum(h * ad_ref[...], axis=1, keepdims=True)


def _tc_proj(x, W, a_s, a_d):
    BN = 1000
    return pl.pallas_call(
        _proj_body,
        grid=(N // BN,),
        in_specs=[pl.BlockSpec((BN, D), lambda i: (i, 0)),
                  pl.BlockSpec((D, D), lambda i: (0, 0)),
                  pl.BlockSpec((1, D), lambda i: (0, 0)),
                  pl.BlockSpec((1, D), lambda i: (0, 0))],
        out_specs=[pl.BlockSpec((BN, D), lambda i: (i, 0)),
                   pl.BlockSpec((BN, 1), lambda i: (i, 0)),
                   pl.BlockSpec((BN, 1), lambda i: (i, 0))],
        out_shape=[jax.ShapeDtypeStruct((N, D), _F32),
                   jax.ShapeDtypeStruct((N, 1), _F32),
                   jax.ShapeDtypeStruct((N, 1), _F32)],
    )(x, W, a_s.reshape(1, D), a_d.reshape(1, D))


def _den1_body(c_ref, dp_ref, deg_ref, sew_ref, asr_ref, ads_ref,
               la_ref, rden_ref):
    c = c_ref[0, 0]
    p = dp_ref[0:80, :] + dp_ref[80:160, :]
    deg = deg_ref[0:80, :] + deg_ref[80:160, :]
    sew = sew_ref[0:80, :] + sew_ref[80:160, :]
    la = sew / jnp.maximum(deg, 1.0)
    tl = asr_ref[...] + ads_ref[...] + c * la
    wl = jnp.exp(jnp.where(tl >= 0, tl, 0.2 * tl))
    la_ref[...] = la
    rden_ref[...] = 1.0 / (p + wl)


def _tc_den1(c, denp, deg, sew, asr_pad, ads_pad):
    full = lambda r: pl.BlockSpec((r, D), lambda: (0, 0))
    return pl.pallas_call(
        _den1_body,
        in_specs=[pl.BlockSpec((1, 1), lambda: (0, 0)),
                  full(160), full(160), full(160), full(80), full(80)],
        out_specs=[full(80), full(80)],
        out_shape=[jax.ShapeDtypeStruct((80, D), _F32),
                   jax.ShapeDtypeStruct((80, D), _F32)],
    )(c.reshape(1, 1), denp, deg, sew, asr_pad, ads_pad)


def _den2_body(c_ref, dp_ref, la_ref, asr_ref, ads_ref, rden_ref):
    c = c_ref[0, 0]
    p = dp_ref[0:80, :] + dp_ref[80:160, :]
    tl = asr_ref[...] + ads_ref[...] + c * la_ref[...]
    wl = jnp.exp(jnp.where(tl >= 0, tl, 0.2 * tl))
    rden_ref[...] = 1.0 / (p + wl)


def _tc_den2(c, denp, la, asr_pad, ads_pad):
    full = lambda r: pl.BlockSpec((r, D), lambda: (0, 0))
    return pl.pallas_call(
        _den2_body,
        in_specs=[pl.BlockSpec((1, 1), lambda: (0, 0)),
                  full(160), full(80), full(80), full(80)],
        out_specs=full(80),
        out_shape=jax.ShapeDtypeStruct((80, D), _F32),
    )(c.reshape(1, 1), denp, la, asr_pad, ads_pad)


def _mid_body(c_ref, p0_ref, p1_ref, h_ref, asr_ref, ads_ref, la_ref,
              rden_ref, w2_ref, as2_ref, ad2_ref, b1_ref,
              g_ref, asr2_ref, ads2_ref):
    c = c_ref[0, 0]
    tl = asr_ref[...] + ads_ref[...] + c * la_ref[...]
    wl = jnp.exp(jnp.where(tl >= 0, tl, 0.2 * tl))
    al = wl * rden_ref[...]
    out1 = p0_ref[...] + p1_ref[...] + al * h_ref[...]
    h2 = out1 + b1_ref[...]
    h2 = jnp.where(h2 > 0, h2, jnp.exp(jnp.minimum(h2, 0.0)) - 1.0)
    g = jnp.dot(h2, w2_ref[...], preferred_element_type=_F32)
    g_ref[...] = g
    asr2_ref[...] = jnp.sum(g * as2_ref[...], axis=1, keepdims=True)
    ads2_ref[...] = jnp.sum(g * ad2_ref[...], axis=1, keepdims=True)


def _tc_mid(c1, p0, p1, h, asr, ads, la, rden, W2, a_s2, a_d2, b1):
    BN = 1000
    vec = lambda: pl.BlockSpec((BN, 1), lambda i: (i, 0))
    mat = lambda: pl.BlockSpec((BN, D), lambda i: (i, 0))
    full = lambda r: pl.BlockSpec((r, D), lambda i: (0, 0))
    return pl.pallas_call(
        _mid_body,
        grid=(N // BN,),
        in_specs=[pl.BlockSpec((1, 1), lambda i: (0, 0)),
                  mat(), mat(), mat(), vec(), vec(), vec(), vec(),
                  full(D), full(1), full(1), full(1)],
        out_specs=[mat(), vec(), vec()],
        out_shape=[jax.ShapeDtypeStruct((N, D), _F32),
                   jax.ShapeDtypeStruct((N, 1), _F32),
                   jax.ShapeDtypeStruct((N, 1), _F32)],
    )(c1.reshape(1, 1), p0, p1, h, asr, ads, la, rden,
      W2, a_s2.reshape(1, D), a_d2.reshape(1, D), b1.reshape(1, D))


def _fin_body(c_ref, p0_ref, p1_ref, g_ref, asr_ref, ads_ref, la_ref,
              rden_ref, b2_ref, o_ref):
    c = c_ref[0, 0]
    tl = asr_ref[...] + ads_ref[...] + c * la_ref[...]
    wl = jnp.exp(jnp.where(tl >= 0, tl, 0.2 * tl))
    al = wl * rden_ref[...]
    o_ref[...] = p0_ref[...] + p1_ref[...] + al * g_ref[...] + b2_ref[...]


def _tc_fin(c2, p0, p1, g, asr2, ads2, la, rden2, b2):
    BN = 1000
    vec = lambda: pl.BlockSpec((BN, 1), lambda i: (i, 0))
    mat = lambda: pl.BlockSpec((BN, D), lambda i: (i, 0))
    return pl.pallas_call(
        _fin_body,
        grid=(N // BN,),
        in_specs=[pl.BlockSpec((1, 1), lambda i: (0, 0)),
                  mat(), mat(), mat(), vec(), vec(), vec(), vec(),
                  pl.BlockSpec((1, D), lambda i: (0, 0))],
        out_specs=mat(),
        out_shape=jax.ShapeDtypeStruct((N, D), _F32),
    )(c2.reshape(1, 1), p0, p1, g, asr2, ads2, la, rden2, b2.reshape(1, D))


_SC_MESH = plsc.VectorSubcoreMesh(core_axis_name="c", subcore_axis_name="s")
_SC_PARAMS = pltpu.CompilerParams(needs_layout_passes=False,
                                  use_tc_tiling_on_sc=False)

_GATHER_DNUMS = lax.GatherDimensionNumbers(
    offset_dims=(), collapsed_slice_dims=(0,), start_index_map=(0,))


def _bcast_lane(v, j):
    idx = jnp.full((16, 1), j, _I32)
    return lax.gather(v, idx, _GATHER_DNUMS, slice_sizes=(1,),
                      mode=lax.GatherScatterMode.PROMISE_IN_BOUNDS)


def _p1_body(src_hbm, dst_hbm, ew_hbm, asr_hbm, ads_hbm, cv_hbm,
             w_hbm, denp_hbm, degp_hbm, sewp_hbm,
             asr_v, ads_v, den_v, deg_v, sew_v,
             src_w, dst_w, ew_w, w_w, cv_v, idx_v,
             den_sh, deg_sh, sew_sh):
    cid = lax.axis_index("c")
    sid = lax.axis_index("s")
    wid = sid * 2 + cid
    pltpu.sync_copy(asr_hbm, asr_v)
    pltpu.sync_copy(ads_hbm, ads_v)
    pltpu.sync_copy(cv_hbm, cv_v)
    pltpu.sync_copy(src_hbm.at[wid], src_w)
    pltpu.sync_copy(dst_hbm.at[wid], dst_w)
    pltpu.sync_copy(ew_hbm.at[wid], ew_w)
    lanes = lax.iota(_I32, 16)
    for j in range(5):
        for k in range(8):
            idx_v[j, pl.ds(k * 16, 16)] = j * 128 + k * 16 + lanes

    def zbody(r, carry):
        z = jnp.zeros((16,), _F32)
        den_v[r, :] = z
        deg_v[r, :] = z
        sew_v[r, :] = z
        return carry
    lax.fori_loop(0, NROW, zbody, 0)

    @pl.when(sid == 0)
    def _():
        pltpu.sync_copy(den_v, den_sh)
        pltpu.sync_copy(deg_v, deg_sh)
        pltpu.sync_copy(sew_v, sew_sh)
    plsc.subcore_barrier()

    cv16 = cv_v[...]
    ones = jnp.ones((16,), _F32)
    ebase = wid * EPW

    def chunk(ch, carry):
        for g in range(8):
            sl = pl.ds(g * 16, 16)
            srcs = src_w[ch, sl]
            dsts = dst_w[ch, sl]
            ewv = ew_w[ch, sl]
            rows = lax.shift_right_logical(dsts, 4)
            cols = jnp.bitwise_and(dsts, 15)
            asrv = plsc.load_gather(
                asr_v, [lax.shift_right_logical(srcs, 4),
                        jnp.bitwise_and(srcs, 15)])
            adsv = plsc.load_gather(ads_v, [rows, cols])
            t = asrv + adsv + cv16 * ewv
            t = jnp.where(t >= 0, t, 0.2 * t)
            w = jnp.exp(t)
            gid = ebase + ch * CH + g * 16 + lanes
            m = gid < E
            w = jnp.where(m, w, 0.0)
            w_w[ch, sl] = w
            plsc.addupdate_scatter(den_v, [rows, cols], w, mask=m)
            plsc.addupdate_scatter(deg_v, [rows, cols], ones, mask=m)
            plsc.addupdate_scatter(sew_v, [rows, cols], ewv, mask=m)
        return carry
    lax.fori_loop(0, NCH, chunk, 0)
    pltpu.sync_copy(w_w, w_hbm.at[wid])
    plsc.subcore_barrier()
    for j in range(5):
        s = pl.ds(j * 128, 128)
        pltpu.sync_copy(den_v.at[s], den_sh.at[idx_v.at[j]], add=True)
        pltpu.sync_copy(deg_v.at[s], deg_sh.at[idx_v.at[j]], add=True)
        pltpu.sync_copy(sew_v.at[s], sew_sh.at[idx_v.at[j]], add=True)
    plsc.subcore_barrier()

    @pl.when(sid == 0)
    def _():
        pltpu.sync_copy(den_sh, denp_hbm.at[cid])
        pltpu.sync_copy(deg_sh, degp_hbm.at[cid])
        pltpu.sync_copy(sew_sh, sewp_hbm.at[cid])


def _sc_pass1(src3, dst3, ew3, asr, ads, cv):
    out = pl.kernel(
        _p1_body,
        out_type=[jax.ShapeDtypeStruct((NW, NCH, CH), _F32),
                  jax.ShapeDtypeStruct((2, NROW, 16), _F32),
                  jax.ShapeDtypeStruct((2, NROW, 16), _F32),
                  jax.ShapeDtypeStruct((2, NROW, 16), _F32)],
        mesh=_SC_MESH,
        compiler_params=_SC_PARAMS,
        scratch_types=[
            pltpu.VMEM((NROW, 16), _F32), pltpu.VMEM((NROW, 16), _F32),
            pltpu.VMEM((NROW, 16), _F32), pltpu.VMEM((NROW, 16), _F32),
            pltpu.VMEM((NROW, 16), _F32),
            pltpu.VMEM((NCH, CH), _I32), pltpu.VMEM((NCH, CH), _I32),
            pltpu.VMEM((NCH, CH), _F32), pltpu.VMEM((NCH, CH), _F32),
            pltpu.VMEM((16,), _F32), pltpu.VMEM((5, 128), _I32),
            pltpu.VMEM_SHARED((NROW, 16), _F32),
            pltpu.VMEM_SHARED((NROW, 16), _F32),
            pltpu.VMEM_SHARED((NROW, 16), _F32),
        ],
    )(src3, dst3, ew3, asr, ads, cv)
    return out


def _p2_body(e_hbm, rden_hbm, h_hbm,
             outp_hbm,
             rden_v, edata, rows2, sem_g,
             out_sh):
    cid = lax.axis_index("c")
    sid = lax.axis_index("s")
    wid = sid * 2 + cid
    pltpu.sync_copy(rden_hbm, rden_v)

    def zb(r, carry):
        for k in range(8):
            rows2[0, r, pl.ds(k * 16, 16)] = jnp.zeros((16,), _F32)
        return carry
    lax.fori_loop(0, CH, zb, 0)
    for t in range(5):
        pltpu.sync_copy(rows2.at[0],
                        out_sh.at[pl.ds(sid * 640 + t * 128, 128)])
    plsc.subcore_barrier()

    pltpu.sync_copy(e_hbm.at[wid, 0], edata.at[0])
    pltpu.async_copy(h_hbm.at[edata.at[0, 0]], rows2.at[0], sem_g.at[0])

    def _scale(b):
        def gb(g, carry2):
            sl = pl.ds(g * 16, 16)
            dsts = edata[b, 1, sl]
            rws = lax.shift_right_logical(dsts, 4)
            cls = jnp.bitwise_and(dsts, 15)
            wv = plsc.bitcast(edata[b, 2, sl], _F32)
            av = wv * plsc.load_gather(rden_v, [rws, cls])
            for j in range(16):
                bc = _bcast_lane(av, j)
                e = g * 16 + j
                for k in range(8):
                    sk = pl.ds(k * 16, 16)
                    rows2[b, e, sk] = rows2[b, e, sk] * bc
            return carry2
        return gb

    def cb(i, carry):
        for b in range(2):
            ch = i * 2 + b
            nb = 1 - b

            @pl.when(ch + 1 < NCH)
            def _(ch=ch, nb=nb):
                pltpu.sync_copy(e_hbm.at[wid, ch + 1], edata.at[nb])
                pltpu.async_copy(h_hbm.at[edata.at[nb, 0]], rows2.at[nb],
                                 sem_g.at[nb])
            pltpu.make_async_copy(h_hbm.at[edata.at[b, 0]], rows2.at[b],
                                  sem_g.at[b]).wait()
            lax.fori_loop(0, 8, _scale(b), 0)
            pltpu.sync_copy(rows2.at[b], out_sh.at[edata.at[b, 1]], add=True)
        return carry
    lax.fori_loop(0, NCH // 2, cb, 0)
    plsc.subcore_barrier()
    s = pl.ds(sid * 640, 640)
    pltpu.sync_copy(out_sh.at[s], outp_hbm.at[cid, s])


def _sc_pass2(edata, rden, h):
    return pl.kernel(
        _p2_body,
        out_type=jax.ShapeDtypeStruct((2, NPAD, D), _F32),
        mesh=_SC_MESH,
        compiler_params=_SC_PARAMS,
        scratch_types=[
            pltpu.VMEM((NROW, 16), _F32),
            pltpu.VMEM((2, 3, CH), _I32),
            pltpu.VMEM((2, CH, D), _F32),
            pltpu.SemaphoreType.DMA((2,)),
            pltpu.VMEM_SHARED((NPAD, D), _F32),
        ],
    )(edata, rden, h)


def kernel(x, edge_index, edge_weight, W1, as1, ad1, We1, ae1, b1,
           W2, as2, ad2, We2, ae2, b2):
    src, dst = edge_index[0], edge_index[1]
    ew = edge_weight[:, 0]
    c1 = jnp.sum(We1.reshape(-1) * ae1.reshape(-1))
    c2 = jnp.sum(We2.reshape(-1) * ae2.reshape(-1))
    cv1 = jnp.full((16,), c1, _F32)
    cv2 = jnp.full((16,), c2, _F32)

    pad_i = jnp.zeros((E_PAD - E,), _I32)
    pad_f = jnp.zeros((E_PAD - E,), _F32)
    src3 = jnp.concatenate([src, pad_i]).reshape(NW, NCH, CH)
    dst3 = jnp.concatenate([dst, pad_i]).reshape(NW, NCH, CH)
    ew3 = jnp.concatenate([ew, pad_f]).reshape(NW, NCH, CH)
    padn = jnp.zeros((NPAD - N,), _F32)

    h, asr1, ads1 = _tc_proj(x, W1, as1, ad1)
    asr1pad = jnp.concatenate([asr1.reshape(N), padn])
    ads1pad = jnp.concatenate([ads1.reshape(N), padn])
    w1, denp1, degp, sewp = _sc_pass1(src3, dst3, ew3,
                                      asr1pad.reshape(NROW, 16),
                                      ads1pad.reshape(NROW, 16), cv1)
    asr1p = asr1pad.reshape(80, D)
    ads1p = ads1pad.reshape(80, D)
    la80, rden1_80 = _tc_den1(c1, denp1.reshape(160, D),
                              degp.reshape(160, D), sewp.reshape(160, D),
                              asr1p, ads1p)
    rden1_640 = rden1_80.reshape(NROW, 16)
    edata1 = jnp.stack(
        [src3, dst3, lax.bitcast_convert_type(w1, _I32)], axis=2)
    outp1 = _sc_pass2(edata1, rden1_640, h)
    la = la80.reshape(NPAD)[:N].reshape(N, 1)
    rden1 = rden1_80.reshape(NPAD)[:N].reshape(N, 1)

    g, asr2, ads2 = _tc_mid(c1, outp1[0, :N], outp1[1, :N], h,
                            asr1, ads1, la, rden1, W2, as2, ad2, b1)

    asr2pad = jnp.concatenate([asr2.reshape(N), padn])
    ads2pad = jnp.concatenate([ads2.reshape(N), padn])
    w2e, denp2, _, _ = _sc_pass1(src3, dst3, ew3,
                                 asr2pad.reshape(NROW, 16),
                                 ads2pad.reshape(NROW, 16), cv2)
    asr2p = asr2pad.reshape(80, D)
    ads2p = ads2pad.reshape(80, D)
    rden2_80 = _tc_den2(c2, denp2.reshape(160, D), la80, asr2p, ads2p)
    edata2 = jnp.stack(
        [src3, dst3, lax.bitcast_convert_type(w2e, _I32)], axis=2)
    outp2 = _sc_pass2(edata2, rden2_80.reshape(NROW, 16), g)
    rden2 = rden2_80.reshape(NPAD)[:N].reshape(N, 1)

    o = _tc_fin(c2, outp2[0, :N], outp2[1, :N], g, asr2, ads2,
                la, rden2, b2)
    return o


# speedup vs baseline: 1.1722x; 1.0018x over previous
import functools

import jax
import jax.numpy as jnp
from jax import lax
from jax.experimental import pallas as pl
from jax.experimental.pallas import tpu as pltpu
from jax.experimental.pallas import tpu_sc as plsc

_F32 = jnp.float32
_I32 = jnp.int32

N = 10000
E = 320000
D = 128
NW = 32
CH = 128
NCH = 80
EPW = NCH * CH
E_PAD = NW * EPW
NROW = 640
NPAD = NROW * 16


def _proj_body(x_ref, w_ref, as_ref, ad_ref, h_ref, asr_ref, ads_ref):
    h = jnp.dot(x_ref[...], w_ref[...], preferred_element_type=_F32)
    h_ref[...] = h
    asr_ref[...] = jnp.sum(h * as_ref[...], axis=1, keepdims=True)
    ads_ref[...] = jnp.sum(h * ad_ref[...], axis=1, keepdims=True)


def _tc_proj(x, W, a_s, a_d):
    BN = 1000
    return pl.pallas_call(
        _proj_body,
        grid=(N // BN,),
        in_specs=[pl.BlockSpec((BN, D), lambda i: (i, 0)),
                  pl.BlockSpec((D, D), lambda i: (0, 0)),
                  pl.BlockSpec((1, D), lambda i: (0, 0)),
                  pl.BlockSpec((1, D), lambda i: (0, 0))],
        out_specs=[pl.BlockSpec((BN, D), lambda i: (i, 0)),
                   pl.BlockSpec((BN, 1), lambda i: (i, 0)),
                   pl.BlockSpec((BN, 1), lambda i: (i, 0))],
        out_shape=[jax.ShapeDtypeStruct((N, D), _F32),
                   jax.ShapeDtypeStruct((N, 1), _F32),
                   jax.ShapeDtypeStruct((N, 1), _F32)],
    )(x, W, a_s.reshape(1, D), a_d.reshape(1, D))


def _den1_body(c_ref, dp_ref, deg_ref, sew_ref, asr_ref, ads_ref,
               la_ref, rden_ref):
    c = c_ref[0, 0]
    p = dp_ref[0:80, :] + dp_ref[80:160, :]
    deg = deg_ref[0:80, :] + deg_ref[80:160, :]
    sew = sew_ref[0:80, :] + sew_ref[80:160, :]
    la = sew / jnp.maximum(deg, 1.0)
    tl = asr_ref[...] + ads_ref[...] + c * la
    wl = jnp.exp(jnp.where(tl >= 0, tl, 0.2 * tl))
    la_ref[...] = la
    rden_ref[...] = 1.0 / (p + wl)


def _tc_den1(c, denp, deg, sew, asr_pad, ads_pad):
    full = lambda r: pl.BlockSpec((r, D), lambda: (0, 0))
    return pl.pallas_call(
        _den1_body,
        in_specs=[pl.BlockSpec((1, 1), lambda: (0, 0)),
                  full(160), full(160), full(160), full(80), full(80)],
        out_specs=[full(80), full(80)],
        out_shape=[jax.ShapeDtypeStruct((80, D), _F32),
                   jax.ShapeDtypeStruct((80, D), _F32)],
    )(c.reshape(1, 1), denp, deg, sew, asr_pad, ads_pad)


def _den2_body(c_ref, dp_ref, la_ref, asr_ref, ads_ref, rden_ref):
    c = c_ref[0, 0]
    p = dp_ref[0:80, :] + dp_ref[80:160, :]
    tl = asr_ref[...] + ads_ref[...] + c * la_ref[...]
    wl = jnp.exp(jnp.where(tl >= 0, tl, 0.2 * tl))
    rden_ref[...] = 1.0 / (p + wl)


def _tc_den2(c, denp, la, asr_pad, ads_pad):
    full = lambda r: pl.BlockSpec((r, D), lambda: (0, 0))
    return pl.pallas_call(
        _den2_body,
        in_specs=[pl.BlockSpec((1, 1), lambda: (0, 0)),
                  full(160), full(80), full(80), full(80)],
        out_specs=full(80),
        out_shape=jax.ShapeDtypeStruct((80, D), _F32),
    )(c.reshape(1, 1), denp, la, asr_pad, ads_pad)


def _mid_body(c_ref, p0_ref, p1_ref, h_ref, asr_ref, ads_ref, la_ref,
              rden_ref, w2_ref, as2_ref, ad2_ref, b1_ref,
              g_ref, asr2_ref, ads2_ref):
    c = c_ref[0, 0]
    tl = asr_ref[...] + ads_ref[...] + c * la_ref[...]
    wl = jnp.exp(jnp.where(tl >= 0, tl, 0.2 * tl))
    al = wl * rden_ref[...]
    out1 = p0_ref[...] + p1_ref[...] + al * h_ref[...]
    h2 = out1 + b1_ref[...]
    h2 = jnp.where(h2 > 0, h2, jnp.exp(jnp.minimum(h2, 0.0)) - 1.0)
    g = jnp.dot(h2, w2_ref[...], preferred_element_type=_F32)
    g_ref[...] = g
    asr2_ref[...] = jnp.sum(g * as2_ref[...], axis=1, keepdims=True)
    ads2_ref[...] = jnp.sum(g * ad2_ref[...], axis=1, keepdims=True)


def _tc_mid(c1, p0, p1, h, asr, ads, la, rden, W2, a_s2, a_d2, b1):
    BN = 1000
    vec = lambda: pl.BlockSpec((BN, 1), lambda i: (i, 0))
    mat = lambda: pl.BlockSpec((BN, D), lambda i: (i, 0))
    full = lambda r: pl.BlockSpec((r, D), lambda i: (0, 0))
    return pl.pallas_call(
        _mid_body,
        grid=(N // BN,),
        in_specs=[pl.BlockSpec((1, 1), lambda i: (0, 0)),
                  mat(), mat(), mat(), vec(), vec(), vec(), vec(),
                  full(D), full(1), full(1), full(1)],
        out_specs=[mat(), vec(), vec()],
        out_shape=[jax.ShapeDtypeStruct((N, D), _F32),
                   jax.ShapeDtypeStruct((N, 1), _F32),
                   jax.ShapeDtypeStruct((N, 1), _F32)],
    )(c1.reshape(1, 1), p0, p1, h, asr, ads, la, rden,
      W2, a_s2.reshape(1, D), a_d2.reshape(1, D), b1.reshape(1, D))


def _fin_body(c_ref, p0_ref, p1_ref, g_ref, asr_ref, ads_ref, la_ref,
              rden_ref, b2_ref, o_ref):
    c = c_ref[0, 0]
    tl = asr_ref[...] + ads_ref[...] + c * la_ref[...]
    wl = jnp.exp(jnp.where(tl >= 0, tl, 0.2 * tl))
    al = wl * rden_ref[...]
    o_ref[...] = p0_ref[...] + p1_ref[...] + al * g_ref[...] + b2_ref[...]


def _tc_fin(c2, p0, p1, g, asr2, ads2, la, rden2, b2):
    BN = 1000
    vec = lambda: pl.BlockSpec((BN, 1), lambda i: (i, 0))
    mat = lambda: pl.BlockSpec((BN, D), lambda i: (i, 0))
    return pl.pallas_call(
        _fin_body,
        grid=(N // BN,),
        in_specs=[pl.BlockSpec((1, 1), lambda i: (0, 0)),
                  mat(), mat(), mat(), vec(), vec(), vec(), vec(),
                  pl.BlockSpec((1, D), lambda i: (0, 0))],
        out_specs=mat(),
        out_shape=jax.ShapeDtypeStruct((N, D), _F32),
    )(c2.reshape(1, 1), p0, p1, g, asr2, ads2, la, rden2, b2.reshape(1, D))


_SC_MESH = plsc.VectorSubcoreMesh(core_axis_name="c", subcore_axis_name="s")
_SC_PARAMS = pltpu.CompilerParams(needs_layout_passes=False,
                                  use_tc_tiling_on_sc=False)

_GATHER_DNUMS = lax.GatherDimensionNumbers(
    offset_dims=(), collapsed_slice_dims=(0,), start_index_map=(0,))


def _bcast_lane(v, j):
    idx = jnp.full((16, 1), j, _I32)
    return lax.gather(v, idx, _GATHER_DNUMS, slice_sizes=(1,),
                      mode=lax.GatherScatterMode.PROMISE_IN_BOUNDS)


def _p1_body(src_hbm, dst_hbm, ew_hbm, asr_hbm, ads_hbm, cv_hbm,
             w_hbm, denp_hbm, degp_hbm, sewp_hbm,
             asr_v, ads_v, den_v, deg_v, sew_v,
             src_w, dst_w, ew_w, w_w, cv_v, idx_v,
             den_sh, deg_sh, sew_sh):
    cid = lax.axis_index("c")
    sid = lax.axis_index("s")
    wid = sid * 2 + cid
    pltpu.sync_copy(asr_hbm, asr_v)
    pltpu.sync_copy(ads_hbm, ads_v)
    pltpu.sync_copy(cv_hbm, cv_v)
    pltpu.sync_copy(src_hbm.at[wid], src_w)
    pltpu.sync_copy(dst_hbm.at[wid], dst_w)
    pltpu.sync_copy(ew_hbm.at[wid], ew_w)
    lanes = lax.iota(_I32, 16)
    for j in range(5):
        for k in range(8):
            idx_v[j, pl.ds(k * 16, 16)] = j * 128 + k * 16 + lanes

    def zbody(r, carry):
        z = jnp.zeros((16,), _F32)
        den_v[r, :] = z
        deg_v[r, :] = z
        sew_v[r, :] = z
        return carry
    lax.fori_loop(0, NROW, zbody, 0)

    @pl.when(sid == 0)
    def _():
        pltpu.sync_copy(den_v, den_sh)
        pltpu.sync_copy(deg_v, deg_sh)
        pltpu.sync_copy(sew_v, sew_sh)
    plsc.subcore_barrier()

    cv16 = cv_v[...]
    ones = jnp.ones((16,), _F32)
    ebase = wid * EPW

    def chunk(ch, carry):
        for g in range(8):
            sl = pl.ds(g * 16, 16)
            srcs = src_w[ch, sl]
            dsts = dst_w[ch, sl]
            ewv = ew_w[ch, sl]
            rows = lax.shift_right_logical(dsts, 4)
            cols = jnp.bitwise_and(dsts, 15)
            asrv = plsc.load_gather(
                asr_v, [lax.shift_right_logical(srcs, 4),
                        jnp.bitwise_and(srcs, 15)])
            adsv = plsc.load_gather(ads_v, [rows, cols])
            t = asrv + adsv + cv16 * ewv
            t = jnp.where(t >= 0, t, 0.2 * t)
            w = jnp.exp(t)
            gid = ebase + ch * CH + g * 16 + lanes
            m = gid < E
            w = jnp.where(m, w, 0.0)
            w_w[ch, sl] = w
            plsc.addupdate_scatter(den_v, [rows, cols], w, mask=m)
            plsc.addupdate_scatter(deg_v, [rows, cols], ones, mask=m)
            plsc.addupdate_scatter(sew_v, [rows, cols], ewv, mask=m)
        return carry
    lax.fori_loop(0, NCH, chunk, 0)
    pltpu.sync_copy(w_w, w_hbm.at[wid])
    plsc.subcore_barrier()
    for j in range(5):
        s = pl.ds(j * 128, 128)
        pltpu.sync_copy(den_v.at[s], den_sh.at[idx_v.at[j]], add=True)
        pltpu.sync_copy(deg_v.at[s], deg_sh.at[idx_v.at[j]], add=True)
        pltpu.sync_copy(sew_v.at[s], sew_sh.at[idx_v.at[j]], add=True)
    plsc.subcore_barrier()

    @pl.when(sid == 0)
    def _():
        pltpu.sync_copy(den_sh, denp_hbm.at[cid])
        pltpu.sync_copy(deg_sh, degp_hbm.at[cid])
        pltpu.sync_copy(sew_sh, sewp_hbm.at[cid])


def _sc_pass1(src3, dst3, ew3, asr, ads, cv):
    out = pl.kernel(
        _p1_body,
        out_type=[jax.ShapeDtypeStruct((NW, NCH, CH), _F32),
                  jax.ShapeDtypeStruct((2, NROW, 16), _F32),
                  jax.ShapeDtypeStruct((2, NROW, 16), _F32),
                  jax.ShapeDtypeStruct((2, NROW, 16), _F32)],
        mesh=_SC_MESH,
        compiler_params=_SC_PARAMS,
        scratch_types=[
            pltpu.VMEM((NROW, 16), _F32), pltpu.VMEM((NROW, 16), _F32),
            pltpu.VMEM((NROW, 16), _F32), pltpu.VMEM((NROW, 16), _F32),
            pltpu.VMEM((NROW, 16), _F32),
            pltpu.VMEM((NCH, CH), _I32), pltpu.VMEM((NCH, CH), _I32),
            pltpu.VMEM((NCH, CH), _F32), pltpu.VMEM((NCH, CH), _F32),
            pltpu.VMEM((16,), _F32), pltpu.VMEM((5, 128), _I32),
            pltpu.VMEM_SHARED((NROW, 16), _F32),
            pltpu.VMEM_SHARED((NROW, 16), _F32),
            pltpu.VMEM_SHARED((NROW, 16), _F32),
        ],
    )(src3, dst3, ew3, asr, ads, cv)
    return out


def _p2_body(e_hbm, rden_hbm, h_hbm,
             outp_hbm,
             rden_v, edata, rows2, sem_g, sem_s,
             out_sh):
    cid = lax.axis_index("c")
    sid = lax.axis_index("s")
    wid = sid * 2 + cid
    pltpu.sync_copy(rden_hbm, rden_v)

    def zb(r, carry):
        for k in range(8):
            rows2[0, r, pl.ds(k * 16, 16)] = jnp.zeros((16,), _F32)
        return carry
    lax.fori_loop(0, CH, zb, 0)
    for t in range(5):
        pltpu.sync_copy(rows2.at[0],
                        out_sh.at[pl.ds(sid * 640 + t * 128, 128)])
    plsc.subcore_barrier()

    pltpu.sync_copy(e_hbm.at[wid, 0], edata.at[0])
    pltpu.async_copy(h_hbm.at[edata.at[0, 0]], rows2.at[0], sem_g.at[0])

    def _scale(b):
        def gb(g, carry2):
            sl = pl.ds(g * 16, 16)
            dsts = edata[b, 1, sl]
            rws = lax.shift_right_logical(dsts, 4)
            cls = jnp.bitwise_and(dsts, 15)
            wv = plsc.bitcast(edata[b, 2, sl], _F32)
            av = wv * plsc.load_gather(rden_v, [rws, cls])
            for j in range(16):
                bc = _bcast_lane(av, j)
                e = g * 16 + j
                for k in range(8):
                    sk = pl.ds(k * 16, 16)
                    rows2[b, e, sk] = rows2[b, e, sk] * bc
            return carry2
        return gb

    def cb(i, carry):
        for b in range(2):
            ch = i * 2 + b
            nb = 1 - b

            @pl.when(ch + 1 < NCH)
            def _(ch=ch, nb=nb):
                @pl.when(ch >= 1)
                def _():
                    pltpu.make_async_copy(
                        rows2.at[nb], out_sh.at[edata.at[nb, 1]],
                        sem_s.at[nb]).wait()
                pltpu.sync_copy(e_hbm.at[wid, ch + 1], edata.at[nb])
                pltpu.async_copy(h_hbm.at[edata.at[nb, 0]], rows2.at[nb],
                                 sem_g.at[nb])
            pltpu.make_async_copy(h_hbm.at[edata.at[b, 0]], rows2.at[b],
                                  sem_g.at[b]).wait()
            lax.fori_loop(0, 8, _scale(b), 0)
            pltpu.async_copy(rows2.at[b], out_sh.at[edata.at[b, 1]],
                             sem_s.at[b], add=True)
        return carry
    lax.fori_loop(0, NCH // 2, cb, 0)
    for b in range(2):
        pltpu.make_async_copy(rows2.at[b], out_sh.at[edata.at[b, 1]],
                              sem_s.at[b]).wait()
    plsc.subcore_barrier()
    s = pl.ds(sid * 640, 640)
    pltpu.sync_copy(out_sh.at[s], outp_hbm.at[cid, s])


def _sc_pass2(edata, rden, h):
    return pl.kernel(
        _p2_body,
        out_type=jax.ShapeDtypeStruct((2, NPAD, D), _F32),
        mesh=_SC_MESH,
        compiler_params=_SC_PARAMS,
        scratch_types=[
            pltpu.VMEM((NROW, 16), _F32),
            pltpu.VMEM((2, 3, CH), _I32),
            pltpu.VMEM((2, CH, D), _F32),
            pltpu.SemaphoreType.DMA((2,)),
            pltpu.SemaphoreType.DMA((2,)),
            pltpu.VMEM_SHARED((NPAD, D), _F32),
        ],
    )(edata, rden, h)


def kernel(x, edge_index, edge_weight, W1, as1, ad1, We1, ae1, b1,
           W2, as2, ad2, We2, ae2, b2):
    src, dst = edge_index[0], edge_index[1]
    ew = edge_weight[:, 0]
    c1 = jnp.sum(We1.reshape(-1) * ae1.reshape(-1))
    c2 = jnp.sum(We2.reshape(-1) * ae2.reshape(-1))
    cv1 = jnp.full((16,), c1, _F32)
    cv2 = jnp.full((16,), c2, _F32)

    pad_i = jnp.zeros((E_PAD - E,), _I32)
    pad_f = jnp.zeros((E_PAD - E,), _F32)
    src3 = jnp.concatenate([src, pad_i]).reshape(NW, NCH, CH)
    dst3 = jnp.concatenate([dst, pad_i]).reshape(NW, NCH, CH)
    ew3 = jnp.concatenate([ew, pad_f]).reshape(NW, NCH, CH)
    padn = jnp.zeros((NPAD - N,), _F32)

    h, asr1, ads1 = _tc_proj(x, W1, as1, ad1)
    asr1pad = jnp.concatenate([asr1.reshape(N), padn])
    ads1pad = jnp.concatenate([ads1.reshape(N), padn])
    w1, denp1, degp, sewp = _sc_pass1(src3, dst3, ew3,
                                      asr1pad.reshape(NROW, 16),
                                      ads1pad.reshape(NROW, 16), cv1)
    asr1p = asr1pad.reshape(80, D)
    ads1p = ads1pad.reshape(80, D)
    la80, rden1_80 = _tc_den1(c1, denp1.reshape(160, D),
                              degp.reshape(160, D), sewp.reshape(160, D),
                              asr1p, ads1p)
    rden1_640 = rden1_80.reshape(NROW, 16)
    edata1 = jnp.stack(
        [src3, dst3, lax.bitcast_convert_type(w1, _I32)], axis=2)
    outp1 = _sc_pass2(edata1, rden1_640, h)
    la = la80.reshape(NPAD)[:N].reshape(N, 1)
    rden1 = rden1_80.reshape(NPAD)[:N].reshape(N, 1)

    g, asr2, ads2 = _tc_mid(c1, outp1[0, :N], outp1[1, :N], h,
                            asr1, ads1, la, rden1, W2, as2, ad2, b1)

    asr2pad = jnp.concatenate([asr2.reshape(N), padn])
    ads2pad = jnp.concatenate([ads2.reshape(N), padn])
    w2e, denp2, _, _ = _sc_pass1(src3, dst3, ew3,
                                 asr2pad.reshape(NROW, 16),
                                 ads2pad.reshape(NROW, 16), cv2)
    asr2p = asr2pad.reshape(80, D)
    ads2p = ads2pad.reshape(80, D)
    rden2_80 = _tc_den2(c2, denp2.reshape(160, D), la80, asr2p, ads2p)
    edata2 = jnp.stack(
        [src3, dst3, lax.bitcast_convert_type(w2e, _I32)], axis=2)
    outp2 = _sc_pass2(edata2, rden2_80.reshape(NROW, 16), g)
    rden2 = rden2_80.reshape(NPAD)[:N].reshape(N, 1)

    o = _tc_fin(c2, outp2[0, :N], outp2[1, :N], g, asr2, ads2,
                la, rden2, b2)
    return o


# speedup vs baseline: 1.3476x; 1.1496x over previous
import functools

import jax
import jax.numpy as jnp
from jax import lax
from jax.experimental import pallas as pl
from jax.experimental.pallas import tpu as pltpu
from jax.experimental.pallas import tpu_sc as plsc

_F32 = jnp.float32
_I32 = jnp.int32

N = 10000
E = 320000
D = 128
NW = 32
CH = 128
NCH = 80
EPW = NCH * CH
E_PAD = NW * EPW
NROW = 640
NPAD = NROW * 16


def _proj_body(x_ref, w_ref, as_ref, ad_ref, h_ref, hb_ref, asr_ref, ads_ref):
    h = jnp.dot(x_ref[...], w_ref[...], preferred_element_type=_F32)
    h_ref[...] = h
    hb_ref[...] = h.astype(jnp.bfloat16)
    asr_ref[...] = jnp.sum(h * as_ref[...], axis=1, keepdims=True)
    ads_ref[...] = jnp.sum(h * ad_ref[...], axis=1, keepdims=True)


def _tc_proj(x, W, a_s, a_d):
    BN = 1000
    return pl.pallas_call(
        _proj_body,
        grid=(N // BN,),
        in_specs=[pl.BlockSpec((BN, D), lambda i: (i, 0)),
                  pl.BlockSpec((D, D), lambda i: (0, 0)),
                  pl.BlockSpec((1, D), lambda i: (0, 0)),
                  pl.BlockSpec((1, D), lambda i: (0, 0))],
        out_specs=[pl.BlockSpec((BN, D), lambda i: (i, 0)),
                   pl.BlockSpec((BN, D), lambda i: (i, 0)),
                   pl.BlockSpec((BN, 1), lambda i: (i, 0)),
                   pl.BlockSpec((BN, 1), lambda i: (i, 0))],
        out_shape=[jax.ShapeDtypeStruct((N, D), _F32),
                   jax.ShapeDtypeStruct((N, D), jnp.bfloat16),
                   jax.ShapeDtypeStruct((N, 1), _F32),
                   jax.ShapeDtypeStruct((N, 1), _F32)],
    )(x, W, a_s.reshape(1, D), a_d.reshape(1, D))


def _den1_body(c_ref, dp_ref, deg_ref, sew_ref, asr_ref, ads_ref,
               la_ref, rden_ref):
    c = c_ref[0, 0]
    p = dp_ref[0:80, :] + dp_ref[80:160, :]
    deg = deg_ref[0:80, :] + deg_ref[80:160, :]
    sew = sew_ref[0:80, :] + sew_ref[80:160, :]
    la = sew / jnp.maximum(deg, 1.0)
    tl = asr_ref[...] + ads_ref[...] + c * la
    wl = jnp.exp(jnp.where(tl >= 0, tl, 0.2 * tl))
    la_ref[...] = la
    rden_ref[...] = 1.0 / (p + wl)


def _tc_den1(c, denp, deg, sew, asr_pad, ads_pad):
    full = lambda r: pl.BlockSpec((r, D), lambda: (0, 0))
    return pl.pallas_call(
        _den1_body,
        in_specs=[pl.BlockSpec((1, 1), lambda: (0, 0)),
                  full(160), full(160), full(160), full(80), full(80)],
        out_specs=[full(80), full(80)],
        out_shape=[jax.ShapeDtypeStruct((80, D), _F32),
                   jax.ShapeDtypeStruct((80, D), _F32)],
    )(c.reshape(1, 1), denp, deg, sew, asr_pad, ads_pad)


def _den2_body(c_ref, dp_ref, la_ref, asr_ref, ads_ref, rden_ref):
    c = c_ref[0, 0]
    p = dp_ref[0:80, :] + dp_ref[80:160, :]
    tl = asr_ref[...] + ads_ref[...] + c * la_ref[...]
    wl = jnp.exp(jnp.where(tl >= 0, tl, 0.2 * tl))
    rden_ref[...] = 1.0 / (p + wl)


def _tc_den2(c, denp, la, asr_pad, ads_pad):
    full = lambda r: pl.BlockSpec((r, D), lambda: (0, 0))
    return pl.pallas_call(
        _den2_body,
        in_specs=[pl.BlockSpec((1, 1), lambda: (0, 0)),
                  full(160), full(80), full(80), full(80)],
        out_specs=full(80),
        out_shape=jax.ShapeDtypeStruct((80, D), _F32),
    )(c.reshape(1, 1), denp, la, asr_pad, ads_pad)


def _mid_body(c_ref, p0_ref, p1_ref, h_ref, asr_ref, ads_ref, la_ref,
              rden_ref, w2_ref, as2_ref, ad2_ref, b1_ref,
              g_ref, gb_ref, asr2_ref, ads2_ref):
    c = c_ref[0, 0]
    tl = asr_ref[...] + ads_ref[...] + c * la_ref[...]
    wl = jnp.exp(jnp.where(tl >= 0, tl, 0.2 * tl))
    al = wl * rden_ref[...]
    out1 = p0_ref[...] + p1_ref[...] + al * h_ref[...]
    h2 = out1 + b1_ref[...]
    h2 = jnp.where(h2 > 0, h2, jnp.exp(jnp.minimum(h2, 0.0)) - 1.0)
    g = jnp.dot(h2, w2_ref[...], preferred_element_type=_F32)
    g_ref[...] = g
    gb_ref[...] = g.astype(jnp.bfloat16)
    asr2_ref[...] = jnp.sum(g * as2_ref[...], axis=1, keepdims=True)
    ads2_ref[...] = jnp.sum(g * ad2_ref[...], axis=1, keepdims=True)


def _tc_mid(c1, p0, p1, h, asr, ads, la, rden, W2, a_s2, a_d2, b1):
    BN = 1000
    vec = lambda: pl.BlockSpec((BN, 1), lambda i: (i, 0))
    mat = lambda: pl.BlockSpec((BN, D), lambda i: (i, 0))
    full = lambda r: pl.BlockSpec((r, D), lambda i: (0, 0))
    return pl.pallas_call(
        _mid_body,
        grid=(N // BN,),
        in_specs=[pl.BlockSpec((1, 1), lambda i: (0, 0)),
                  mat(), mat(), mat(), vec(), vec(), vec(), vec(),
                  full(D), full(1), full(1), full(1)],
        out_specs=[mat(), mat(), vec(), vec()],
        out_shape=[jax.ShapeDtypeStruct((N, D), _F32),
                   jax.ShapeDtypeStruct((N, D), jnp.bfloat16),
                   jax.ShapeDtypeStruct((N, 1), _F32),
                   jax.ShapeDtypeStruct((N, 1), _F32)],
    )(c1.reshape(1, 1), p0, p1, h, asr, ads, la, rden,
      W2, a_s2.reshape(1, D), a_d2.reshape(1, D), b1.reshape(1, D))


def _fin_body(c_ref, p0_ref, p1_ref, g_ref, asr_ref, ads_ref, la_ref,
              rden_ref, b2_ref, o_ref):
    c = c_ref[0, 0]
    tl = asr_ref[...] + ads_ref[...] + c * la_ref[...]
    wl = jnp.exp(jnp.where(tl >= 0, tl, 0.2 * tl))
    al = wl * rden_ref[...]
    o_ref[...] = p0_ref[...] + p1_ref[...] + al * g_ref[...] + b2_ref[...]


def _tc_fin(c2, p0, p1, g, asr2, ads2, la, rden2, b2):
    BN = 1000
    vec = lambda: pl.BlockSpec((BN, 1), lambda i: (i, 0))
    mat = lambda: pl.BlockSpec((BN, D), lambda i: (i, 0))
    return pl.pallas_call(
        _fin_body,
        grid=(N // BN,),
        in_specs=[pl.BlockSpec((1, 1), lambda i: (0, 0)),
                  mat(), mat(), mat(), vec(), vec(), vec(), vec(),
                  pl.BlockSpec((1, D), lambda i: (0, 0))],
        out_specs=mat(),
        out_shape=jax.ShapeDtypeStruct((N, D), _F32),
    )(c2.reshape(1, 1), p0, p1, g, asr2, ads2, la, rden2, b2.reshape(1, D))


_SC_MESH = plsc.VectorSubcoreMesh(core_axis_name="c", subcore_axis_name="s")
_SC_PARAMS = pltpu.CompilerParams(needs_layout_passes=False,
                                  use_tc_tiling_on_sc=False)

_GATHER_DNUMS = lax.GatherDimensionNumbers(
    offset_dims=(), collapsed_slice_dims=(0,), start_index_map=(0,))


def _bcast_lane(v, j):
    idx = jnp.full((16, 1), j, _I32)
    return lax.gather(v, idx, _GATHER_DNUMS, slice_sizes=(1,),
                      mode=lax.GatherScatterMode.PROMISE_IN_BOUNDS)


def _p1_body(src_hbm, dst_hbm, ew_hbm, asr_hbm, ads_hbm, cv_hbm,
             w_hbm, denp_hbm, degp_hbm, sewp_hbm,
             asr_v, ads_v, den_v, deg_v, sew_v,
             src_w, dst_w, ew_w, w_w, cv_v, idx_v,
             den_sh, deg_sh, sew_sh):
    cid = lax.axis_index("c")
    sid = lax.axis_index("s")
    wid = sid * 2 + cid
    pltpu.sync_copy(asr_hbm, asr_v)
    pltpu.sync_copy(ads_hbm, ads_v)
    pltpu.sync_copy(cv_hbm, cv_v)
    pltpu.sync_copy(src_hbm.at[wid], src_w)
    pltpu.sync_copy(dst_hbm.at[wid], dst_w)
    pltpu.sync_copy(ew_hbm.at[wid], ew_w)
    lanes = lax.iota(_I32, 16)
    for j in range(5):
        for k in range(8):
            idx_v[j, pl.ds(k * 16, 16)] = j * 128 + k * 16 + lanes

    def zbody(r, carry):
        z = jnp.zeros((16,), _F32)
        den_v[r, :] = z
        deg_v[r, :] = z
        sew_v[r, :] = z
        return carry
    lax.fori_loop(0, NROW, zbody, 0)

    @pl.when(sid == 0)
    def _():
        pltpu.sync_copy(den_v, den_sh)
        pltpu.sync_copy(deg_v, deg_sh)
        pltpu.sync_copy(sew_v, sew_sh)
    plsc.subcore_barrier()

    cv16 = cv_v[...]
    ones = jnp.ones((16,), _F32)
    ebase = wid * EPW

    def chunk(ch, carry):
        for g in range(8):
            sl = pl.ds(g * 16, 16)
            srcs = src_w[ch, sl]
            dsts = dst_w[ch, sl]
            ewv = ew_w[ch, sl]
            rows = lax.shift_right_logical(dsts, 4)
            cols = jnp.bitwise_and(dsts, 15)
            asrv = plsc.load_gather(
                asr_v, [lax.shift_right_logical(srcs, 4),
                        jnp.bitwise_and(srcs, 15)])
            adsv = plsc.load_gather(ads_v, [rows, cols])
            t = asrv + adsv + cv16 * ewv
            t = jnp.where(t >= 0, t, 0.2 * t)
            w = jnp.exp(t)
            gid = ebase + ch * CH + g * 16 + lanes
            m = gid < E
            w = jnp.where(m, w, 0.0)
            w_w[ch, sl] = w
            plsc.addupdate_scatter(den_v, [rows, cols], w, mask=m)
            plsc.addupdate_scatter(deg_v, [rows, cols], ones, mask=m)
            plsc.addupdate_scatter(sew_v, [rows, cols], ewv, mask=m)
        return carry
    lax.fori_loop(0, NCH, chunk, 0)
    pltpu.sync_copy(w_w, w_hbm.at[wid])
    plsc.subcore_barrier()
    for j in range(5):
        s = pl.ds(j * 128, 128)
        pltpu.sync_copy(den_v.at[s], den_sh.at[idx_v.at[j]], add=True)
        pltpu.sync_copy(deg_v.at[s], deg_sh.at[idx_v.at[j]], add=True)
        pltpu.sync_copy(sew_v.at[s], sew_sh.at[idx_v.at[j]], add=True)
    plsc.subcore_barrier()

    @pl.when(sid == 0)
    def _():
        pltpu.sync_copy(den_sh, denp_hbm.at[cid])
        pltpu.sync_copy(deg_sh, degp_hbm.at[cid])
        pltpu.sync_copy(sew_sh, sewp_hbm.at[cid])


def _sc_pass1(src3, dst3, ew3, asr, ads, cv):
    out = pl.kernel(
        _p1_body,
        out_type=[jax.ShapeDtypeStruct((NW, NCH, CH), _F32),
                  jax.ShapeDtypeStruct((2, NROW, 16), _F32),
                  jax.ShapeDtypeStruct((2, NROW, 16), _F32),
                  jax.ShapeDtypeStruct((2, NROW, 16), _F32)],
        mesh=_SC_MESH,
        compiler_params=_SC_PARAMS,
        scratch_types=[
            pltpu.VMEM((NROW, 16), _F32), pltpu.VMEM((NROW, 16), _F32),
            pltpu.VMEM((NROW, 16), _F32), pltpu.VMEM((NROW, 16), _F32),
            pltpu.VMEM((NROW, 16), _F32),
            pltpu.VMEM((NCH, CH), _I32), pltpu.VMEM((NCH, CH), _I32),
            pltpu.VMEM((NCH, CH), _F32), pltpu.VMEM((NCH, CH), _F32),
            pltpu.VMEM((16,), _F32), pltpu.VMEM((5, 128), _I32),
            pltpu.VMEM_SHARED((NROW, 16), _F32),
            pltpu.VMEM_SHARED((NROW, 16), _F32),
            pltpu.VMEM_SHARED((NROW, 16), _F32),
        ],
    )(src3, dst3, ew3, asr, ads, cv)
    return out


_MSK_HI = -65536


def _p2_body(e_hbm, rden_hbm, h_hbm,
             outp_hbm,
             rden_v, edata, rows_bf, rows_f, sem_g,
             out_sh):
    cid = lax.axis_index("c")
    sid = lax.axis_index("s")
    wid = sid * 2 + cid
    pltpu.sync_copy(rden_hbm, rden_v)

    def zb(r, carry):
        for k in range(8):
            rows_f[r, pl.ds(k * 16, 16)] = jnp.zeros((16,), _F32)
        return carry
    lax.fori_loop(0, CH, zb, 0)
    for t in range(5):
        pltpu.sync_copy(rows_f,
                        out_sh.at[pl.ds(sid * 640 + t * 128, 128)])
    plsc.subcore_barrier()

    pltpu.sync_copy(e_hbm.at[wid, 0], edata.at[0])
    pltpu.async_copy(h_hbm.at[edata.at[0, 0]], rows_bf.at[0], sem_g.at[0])

    iota16 = lax.iota(_I32, 16)

    def _scale(b):
        def gb(g, carry2):
            sl = pl.ds(g * 16, 16)
            dsts = edata[b, 1, sl]
            rws = lax.shift_right_logical(dsts, 4)
            cls = jnp.bitwise_and(dsts, 15)
            wv = plsc.bitcast(edata[b, 2, sl], _F32)
            av = wv * plsc.load_gather(rden_v, [rws, cls])
            for j in range(16):
                bc = _bcast_lane(av, j)
                e = g * 16 + j
                erow = jnp.zeros((16,), _I32) + e
                for k in range(4):
                    w32 = rows_bf[b, e, pl.ds(k * 16, 16)]
                    fe = plsc.bitcast(lax.shift_left(w32, 16), _F32) * bc
                    fo = plsc.bitcast(jnp.bitwise_and(w32, _MSK_HI),
                                      _F32) * bc
                    ce = k * 32 + 2 * iota16
                    plsc.store_scatter(rows_f, [erow, ce], fe)
                    plsc.store_scatter(rows_f, [erow, ce + 1], fo)
            return carry2
        return gb

    def cb(i, carry):
        for b in range(2):
            ch = i * 2 + b
            nb = 1 - b

            @pl.when(ch + 1 < NCH)
            def _(ch=ch, nb=nb):
                pltpu.sync_copy(e_hbm.at[wid, ch + 1], edata.at[nb])
                pltpu.async_copy(h_hbm.at[edata.at[nb, 0]], rows_bf.at[nb],
                                 sem_g.at[nb])
            pltpu.make_async_copy(h_hbm.at[edata.at[b, 0]], rows_bf.at[b],
                                  sem_g.at[b]).wait()
            lax.fori_loop(0, 8, _scale(b), 0)
            pltpu.sync_copy(rows_f, out_sh.at[edata.at[b, 1]], add=True)
        return carry
    lax.fori_loop(0, NCH // 2, cb, 0)
    plsc.subcore_barrier()
    s = pl.ds(sid * 640, 640)
    pltpu.sync_copy(out_sh.at[s], outp_hbm.at[cid, s])


def _sc_pass2(edata, rden, hb_i32):
    return pl.kernel(
        _p2_body,
        out_type=jax.ShapeDtypeStruct((2, NPAD, D), _F32),
        mesh=_SC_MESH,
        compiler_params=_SC_PARAMS,
        scratch_types=[
            pltpu.VMEM((NROW, 16), _F32),
            pltpu.VMEM((2, 3, CH), _I32),
            pltpu.VMEM((2, CH, D // 2), _I32),
            pltpu.VMEM((CH, D), _F32),
            pltpu.SemaphoreType.DMA((2,)),
            pltpu.VMEM_SHARED((NPAD, D), _F32),
        ],
    )(edata, rden, hb_i32)


def kernel(x, edge_index, edge_weight, W1, as1, ad1, We1, ae1, b1,
           W2, as2, ad2, We2, ae2, b2):
    src, dst = edge_index[0], edge_index[1]
    ew = edge_weight[:, 0]
    c1 = jnp.sum(We1.reshape(-1) * ae1.reshape(-1))
    c2 = jnp.sum(We2.reshape(-1) * ae2.reshape(-1))
    cv1 = jnp.full((16,), c1, _F32)
    cv2 = jnp.full((16,), c2, _F32)

    pad_i = jnp.zeros((E_PAD - E,), _I32)
    pad_f = jnp.zeros((E_PAD - E,), _F32)
    src3 = jnp.concatenate([src, pad_i]).reshape(NW, NCH, CH)
    dst3 = jnp.concatenate([dst, pad_i]).reshape(NW, NCH, CH)
    ew3 = jnp.concatenate([ew, pad_f]).reshape(NW, NCH, CH)
    padn = jnp.zeros((NPAD - N,), _F32)

    h, hb, asr1, ads1 = _tc_proj(x, W1, as1, ad1)
    hb_i32 = lax.bitcast_convert_type(hb.reshape(N, D // 2, 2), _I32)
    asr1pad = jnp.concatenate([asr1.reshape(N), padn])
    ads1pad = jnp.concatenate([ads1.reshape(N), padn])
    w1, denp1, degp, sewp = _sc_pass1(src3, dst3, ew3,
                                      asr1pad.reshape(NROW, 16),
                                      ads1pad.reshape(NROW, 16), cv1)
    asr1p = asr1pad.reshape(80, D)
    ads1p = ads1pad.reshape(80, D)
    la80, rden1_80 = _tc_den1(c1, denp1.reshape(160, D),
                              degp.reshape(160, D), sewp.reshape(160, D),
                              asr1p, ads1p)
    rden1_640 = rden1_80.reshape(NROW, 16)
    edata1 = jnp.stack(
        [src3, dst3, lax.bitcast_convert_type(w1, _I32)], axis=2)
    outp1 = _sc_pass2(edata1, rden1_640, hb_i32)
    la = la80.reshape(NPAD)[:N].reshape(N, 1)
    rden1 = rden1_80.reshape(NPAD)[:N].reshape(N, 1)

    g, gb, asr2, ads2 = _tc_mid(c1, outp1[0, :N], outp1[1, :N], h,
                                asr1, ads1, la, rden1, W2, as2, ad2, b1)
    gb_i32 = lax.bitcast_convert_type(gb.reshape(N, D // 2, 2), _I32)

    asr2pad = jnp.concatenate([asr2.reshape(N), padn])
    ads2pad = jnp.concatenate([ads2.reshape(N), padn])
    w2e, denp2, _, _ = _sc_pass1(src3, dst3, ew3,
                                 asr2pad.reshape(NROW, 16),
                                 ads2pad.reshape(NROW, 16), cv2)
    asr2p = asr2pad.reshape(80, D)
    ads2p = ads2pad.reshape(80, D)
    rden2_80 = _tc_den2(c2, denp2.reshape(160, D), la80, asr2p, ads2p)
    edata2 = jnp.stack(
        [src3, dst3, lax.bitcast_convert_type(w2e, _I32)], axis=2)
    outp2 = _sc_pass2(edata2, rden2_80.reshape(NROW, 16), gb_i32)
    rden2 = rden2_80.reshape(NPAD)[:N].reshape(N, 1)

    o = _tc_fin(c2, outp2[0, :N], outp2[1, :N], g, asr2, ads2,
                la, rden2, b2)
    return o
